# baseline (speedup 1.0000x reference)
import functools

import jax
import jax.numpy as jnp
from jax import lax
from jax.experimental import pallas as pl
from jax.experimental.pallas import tpu as pltpu
from jax.experimental.pallas import tpu_sc as plsc

_C = 1000
_CW = _C // 4
_B = 16384
_NC = 2
_NS = 16
_NW = _NC * _NS
_RPW = _B // _NW
_CHUNK = 64
_NCHUNK = _RPW // _CHUNK
_L = 16
_GROUPS = _CHUNK // _L
_NBUF = 2


def _onehot_body(labels_hbm, zeros_hbm, out_hbm, lbl_v, buf0, buf1,
                 sem0, sem1):
    cid = lax.axis_index("c")
    sid = lax.axis_index("s")
    wid = sid * _NC + cid
    base = wid * _RPW

    bufs = [buf0, buf1]
    sems = [sem0, sem1]

    pltpu.sync_copy(labels_hbm.at[pl.ds(base, _RPW)], lbl_v)
    zfill = []
    for b in range(_NBUF):
        d = pltpu.make_async_copy(zeros_hbm, bufs[b], sems[b])
        d.start()
        zfill.append(d)

    zeros_v = jnp.zeros((_L,), jnp.int32)
    lane_v = lax.iota(jnp.int32, _L)
    one_v = jnp.ones((_L,), jnp.int32)
    two_v = jnp.full((_L,), 2, jnp.int32)
    eight_v = jnp.full((_L,), 8, jnp.int32)
    three_v = jnp.full((_L,), 3, jnp.int32)

    def scatter(g, buf, on):
        row0 = g * _CHUNK
        for j in range(_GROUPS):
            rows = lane_v + (j * _L)
            labels = lbl_v[pl.ds(row0 + j * _L, _L)]
            words = lax.shift_right_logical(labels, two_v)
            if on:
                val = lax.shift_left(
                    one_v, lax.mul(eight_v, lax.bitwise_and(labels, three_v)))
            else:
                val = zeros_v
            plsc.store_scatter(buf, [rows, words], val)

    copies = [None] * _NCHUNK
    for g in range(_NCHUNK):
        b = g % _NBUF
        if g < _NBUF:
            zfill[b].wait()
        else:
            copies[g - _NBUF].wait()
            scatter(g - _NBUF, bufs[b], False)
        scatter(g, bufs[b], True)
        d = pltpu.make_async_copy(
            bufs[b],
            out_hbm.at[pl.ds(base + g * _CHUNK, _CHUNK), :],
            sems[b],
        )
        d.start()
        copies[g] = d
    for g in range(_NCHUNK - _NBUF, _NCHUNK):
        copies[g].wait()


@jax.jit
def kernel(labels):
    labels = labels.astype(jnp.int32)
    zeros_block = jnp.zeros((_CHUNK, _CW), jnp.int32)
    mesh = plsc.VectorSubcoreMesh(core_axis_name="c", subcore_axis_name="s")
    run = functools.partial(
        pl.kernel,
        out_type=jax.ShapeDtypeStruct((_B, _CW), jnp.int32),
        mesh=mesh,
        scratch_types=[
            pltpu.VMEM((_RPW,), jnp.int32),
            pltpu.VMEM((_CHUNK, _CW), jnp.int32),
            pltpu.VMEM((_CHUNK, _CW), jnp.int32),
            pltpu.SemaphoreType.DMA,
            pltpu.SemaphoreType.DMA,
        ],
        compiler_params=pltpu.CompilerParams(needs_layout_passes=False),
    )(_onehot_body)
    packed = run(labels, zeros_block)
    bytes_ = jax.lax.bitcast_convert_type(packed, jnp.int8)
    return bytes_.reshape(_B, _C).astype(jnp.int32)


# speedup vs baseline: 1.7580x; 1.7580x over previous
import functools

import jax
import jax.numpy as jnp
from jax import lax
from jax.experimental import pallas as pl
from jax.experimental.pallas import tpu as pltpu
from jax.experimental.pallas import tpu_sc as plsc

_C = 1000
_B = 16384
_NC = 2
_NS = 16
_NW = _NC * _NS
_RPW = _B // _NW
_CHUNK = 32
_NCHUNK = _RPW // _CHUNK
_L = 16
_GROUPS = _CHUNK // _L
_NBUF = 2


def _onehot_body(labels_hbm, zeros_hbm, out_hbm, lbl_v, buf0, buf1,
                 sem0, sem1):
    cid = lax.axis_index("c")
    sid = lax.axis_index("s")
    wid = sid * _NC + cid
    base = wid * _RPW

    bufs = [buf0, buf1]
    sems = [sem0, sem1]

    pltpu.sync_copy(labels_hbm.at[pl.ds(base, _RPW)], lbl_v)
    zfill = []
    for b in range(_NBUF):
        d = pltpu.make_async_copy(zeros_hbm, bufs[b], sems[b])
        d.start()
        zfill.append(d)

    ones_v = jnp.ones((_L,), jnp.int32)
    zeros_v = jnp.zeros((_L,), jnp.int32)
    lane_v = lax.iota(jnp.int32, _L)

    def scatter(g, buf, val):
        row0 = g * _CHUNK
        for j in range(_GROUPS):
            rows = lane_v + (j * _L)
            cols = lbl_v[pl.ds(row0 + j * _L, _L)]
            plsc.store_scatter(buf, [rows, cols], val)

    copies = [None] * _NCHUNK
    for g in range(_NCHUNK):
        b = g % _NBUF
        if g < _NBUF:
            zfill[b].wait()
        else:
            copies[g - _NBUF].wait()
            scatter(g - _NBUF, bufs[b], zeros_v)
        scatter(g, bufs[b], ones_v)
        d = pltpu.make_async_copy(
            bufs[b],
            out_hbm.at[pl.ds(base + g * _CHUNK, _CHUNK), :],
            sems[b],
        )
        d.start()
        copies[g] = d
    for g in range(_NCHUNK - _NBUF, _NCHUNK):
        copies[g].wait()


@jax.jit
def kernel(labels):
    labels = labels.astype(jnp.int32)
    zeros_block = jnp.zeros((_CHUNK, _C), jnp.int32)
    mesh = plsc.VectorSubcoreMesh(core_axis_name="c", subcore_axis_name="s")
    run = functools.partial(
        pl.kernel,
        out_type=jax.ShapeDtypeStruct((_B, _C), jnp.int32),
        mesh=mesh,
        scratch_types=[
            pltpu.VMEM((_RPW,), jnp.int32),
            pltpu.VMEM((_CHUNK, _C), jnp.int32),
            pltpu.VMEM((_CHUNK, _C), jnp.int32),
            pltpu.SemaphoreType.DMA,
            pltpu.SemaphoreType.DMA,
        ],
        compiler_params=pltpu.CompilerParams(needs_layout_passes=False),
    )(_onehot_body)
    return run(labels, zeros_block)


# speedup vs baseline: 1.7745x; 1.0094x over previous
import functools

import jax
import jax.numpy as jnp
from jax import lax
from jax.experimental import pallas as pl
from jax.experimental.pallas import tpu as pltpu
from jax.experimental.pallas import tpu_sc as plsc

_C = 1000
_B = 16384
_NC = 2
_NS = 16
_NW = _NC * _NS
_RPW = _B // _NW
_CHUNK = 16
_NCHUNK = _RPW // _CHUNK
_L = 16
_GROUPS = _CHUNK // _L
_NBUF = 2


def _onehot_body(labels_hbm, zeros_hbm, out_hbm, lbl_v, buf0, buf1,
                 sem0, sem1):
    cid = lax.axis_index("c")
    sid = lax.axis_index("s")
    wid = sid * _NC + cid
    base = wid * _RPW

    bufs = [buf0, buf1]
    sems = [sem0, sem1]

    pltpu.sync_copy(labels_hbm.at[pl.ds(base, _RPW)], lbl_v)
    zfill = []
    for b in range(_NBUF):
        d = pltpu.make_async_copy(zeros_hbm, bufs[b], sems[b])
        d.start()
        zfill.append(d)

    ones_v = jnp.ones((_L,), jnp.int32)
    zeros_v = jnp.zeros((_L,), jnp.int32)
    lane_v = lax.iota(jnp.int32, _L)

    def scatter(g, buf, val):
        row0 = g * _CHUNK
        for j in range(_GROUPS):
            rows = lane_v + (j * _L)
            cols = lbl_v[pl.ds(row0 + j * _L, _L)]
            plsc.store_scatter(buf, [rows, cols], val)

    copies = [None] * _NCHUNK
    for g in range(_NCHUNK):
        b = g % _NBUF
        if g < _NBUF:
            zfill[b].wait()
        else:
            copies[g - _NBUF].wait()
            scatter(g - _NBUF, bufs[b], zeros_v)
        scatter(g, bufs[b], ones_v)
        d = pltpu.make_async_copy(
            bufs[b],
            out_hbm.at[pl.ds(base + g * _CHUNK, _CHUNK), :],
            sems[b],
        )
        d.start()
        copies[g] = d
    for g in range(_NCHUNK - _NBUF, _NCHUNK):
        copies[g].wait()


@jax.jit
def kernel(labels):
    labels = labels.astype(jnp.int32)
    zeros_block = jnp.zeros((_CHUNK, _C), jnp.int32)
    mesh = plsc.VectorSubcoreMesh(core_axis_name="c", subcore_axis_name="s")
    run = functools.partial(
        pl.kernel,
        out_type=jax.ShapeDtypeStruct((_B, _C), jnp.int32),
        mesh=mesh,
        scratch_types=[
            pltpu.VMEM((_RPW,), jnp.int32),
            pltpu.VMEM((_CHUNK, _C), jnp.int32),
            pltpu.VMEM((_CHUNK, _C), jnp.int32),
            pltpu.SemaphoreType.DMA,
            pltpu.SemaphoreType.DMA,
        ],
        compiler_params=pltpu.CompilerParams(needs_layout_passes=False),
    )(_onehot_body)
    return run(labels, zeros_block)
